# baseline (speedup 1.0000x reference)
import functools

import jax
import jax.numpy as jnp
from jax import lax
from jax.experimental import pallas as pl
from jax.experimental.pallas import tpu as pltpu
from jax.experimental.pallas import tpu_sc as plsc

BATCH = 16384
FACTORS = 64
NUM_CORES = 2
NUM_SUBCORES = 16
NUM_WORKERS = NUM_CORES * NUM_SUBCORES
N_SC = 12288
N_TC = BATCH - N_SC
BPW = N_SC // NUM_WORKERS
IDX_CHUNK = 128
NCHUNK = BPW // IDX_CHUNK
NGROUPS = BPW // 16
NSUB = 8

_mesh = plsc.VectorSubcoreMesh(core_axis_name="c", subcore_axis_name="s")


@functools.partial(
    pl.kernel,
    out_type=jax.ShapeDtypeStruct((N_SC,), jnp.float32),
    mesh=_mesh,
    compiler_params=pltpu.CompilerParams(needs_layout_passes=False),
    scratch_types=[
        pltpu.VMEM((NCHUNK, IDX_CHUNK), jnp.int32),
        pltpu.VMEM((NCHUNK, IDX_CHUNK), jnp.int32),
        pltpu.VMEM((3, 2, FACTORS, 128), jnp.float32),
        pltpu.VMEM((3, 2, FACTORS, 128), jnp.float32),
        pltpu.VMEM((BPW,), jnp.float32),
        pltpu.SemaphoreType.DMA,
        pltpu.SemaphoreType.DMA,
        pltpu.SemaphoreType.DMA,
        pltpu.SemaphoreType.DMA,
    ],
)
def _bpr_sc(users_hbm, item_hbm, utab_hbm, itab_hbm, out_hbm,
            uidx_v, iidx_v, uwin_v, iwin_v, out_v, sem_idx,
            sem_a, sem_b, sem_c):
    wid = lax.axis_index("s") * NUM_CORES + lax.axis_index("c")
    base = wid * BPW

    for k in range(NCHUNK):
        pltpu.async_copy(users_hbm.at[pl.ds(base + k * IDX_CHUNK, IDX_CHUNK)],
                         uidx_v.at[k], sem_idx)
        pltpu.async_copy(item_hbm.at[pl.ds(base + k * IDX_CHUNK, IDX_CHUNK)],
                         iidx_v.at[k], sem_idx)
    for k in range(NCHUNK):
        pltpu.make_async_copy(users_hbm.at[pl.ds(base, IDX_CHUNK)],
                              uidx_v.at[k], sem_idx).wait()
        pltpu.make_async_copy(item_hbm.at[pl.ds(base, IDX_CHUNK)],
                              iidx_v.at[k], sem_idx).wait()

    lane = lax.iota(jnp.int32, 16)
    sems = (sem_a, sem_b, sem_c)

    def _hsum(v):
        for s in (8, 4, 2, 1):
            v = v + jnp.take_along_axis(v, lane ^ s, axis=0,
                                        mode="promise_in_bounds")
        return v

    def group_body(a, carry):
        uidx16 = uidx_v[a // 8, pl.ds((a % 8) * 16, 16)]
        iidx16 = iidx_v[a // 8, pl.ds((a % 8) * 16, 16)]
        ucol = (uidx16 >> 7) << 7
        icol = (iidx16 >> 7) << 7
        uoff = uidx16 & 127
        ioff = iidx16 & 127

        def fire(sub):
            slot = sub % 3
            for jj in range(2):
                j = sub * 2 + jj
                uc = pl.multiple_of(ucol[j], 128)
                ic = pl.multiple_of(icol[j], 128)
                pltpu.async_copy(utab_hbm.at[:, pl.ds(uc, 128)],
                                 uwin_v.at[slot, jj], sems[slot])
                pltpu.async_copy(itab_hbm.at[:, pl.ds(ic, 128)],
                                 iwin_v.at[slot, jj], sems[slot])

        def wait(sub):
            slot = sub % 3
            for jj in range(2):
                pltpu.make_async_copy(utab_hbm.at[:, pl.ds(0, 128)],
                                      uwin_v.at[slot, jj], sems[slot]).wait()
                pltpu.make_async_copy(itab_hbm.at[:, pl.ds(0, 128)],
                                      iwin_v.at[slot, jj], sems[slot]).wait()

        out16 = jnp.zeros((16,), jnp.float32)
        fire(0)
        fire(1)
        for sub in range(NSUB):
            if sub + 2 < NSUB:
                fire(sub + 2)
            wait(sub)
            slot = sub % 3
            for jj in range(2):
                j = sub * 2 + jj
                ul = jnp.full((16,), uoff[j], jnp.int32)
                il = jnp.full((16,), ioff[j], jnp.int32)
                acc = None
                for c in range(FACTORS // 16):
                    fidx = lane + (c * 16)
                    ug = plsc.load_gather(uwin_v.at[slot, jj], [fidx, ul])
                    ig = plsc.load_gather(iwin_v.at[slot, jj], [fidx, il])
                    p = ug * ig
                    acc = p if acc is None else acc + p
                out16 = jnp.where(lane == j, _hsum(acc), out16)
        out_v[pl.ds(a * 16, 16)] = out16
        return carry

    lax.fori_loop(0, NGROUPS, group_body, 0, unroll=False)

    pltpu.sync_copy(out_v, out_hbm.at[pl.ds(base, BPW)])


def _bpr_tc_body(uidx_s, iidx_s, utab_hbm, itab_hbm, out_ref,
                 uwin_v, iwin_v, *sems):
    j128 = lax.broadcasted_iota(jnp.int32, (1, 128), 1)
    nsub = 64
    nslot = 12

    def block_body(a, carry):
        def fire(sub):
            slot = sub % nslot
            for jj in range(2):
                r = a * 128 + sub * 2 + jj
                u = uidx_s[r]
                i = iidx_s[r]
                uc = pl.multiple_of((u >> 7) << 7, 128)
                ic = pl.multiple_of((i >> 7) << 7, 128)
                pltpu.make_async_copy(utab_hbm.at[:, pl.ds(uc, 128)],
                                      uwin_v.at[slot, jj], sems[slot]).start()
                pltpu.make_async_copy(itab_hbm.at[:, pl.ds(ic, 128)],
                                      iwin_v.at[slot, jj], sems[slot]).start()

        def wait(sub):
            slot = sub % nslot
            for jj in range(2):
                pltpu.make_async_copy(utab_hbm.at[:, pl.ds(0, 128)],
                                      uwin_v.at[slot, jj], sems[slot]).wait()
                pltpu.make_async_copy(itab_hbm.at[:, pl.ds(0, 128)],
                                      iwin_v.at[slot, jj], sems[slot]).wait()

        out128 = jnp.zeros((1, 128), jnp.float32)
        for s in range(nslot - 1):
            fire(s)
        for sub in range(nsub):
            if sub + nslot - 1 < nsub:
                fire(sub + nslot - 1)
            wait(sub)
            slot = sub % nslot
            for jj in range(2):
                r = a * 128 + sub * 2 + jj
                j = sub * 2 + jj
                cu = uidx_s[r] & 127
                ci = iidx_s[r] & 127
                wu = uwin_v[slot, jj]
                wi = iwin_v[slot, jj]
                gu = jnp.take_along_axis(
                    wu, jnp.full((FACTORS, 1), cu, jnp.int32), axis=1)
                gi = jnp.take_along_axis(
                    wi, jnp.full((FACTORS, 1), ci, jnp.int32), axis=1)
                s = jnp.sum(gu * gi)
                out128 = jnp.where(j128 == j, s, out128)
        base = pl.multiple_of(a * 128, 128)
        out_ref[pl.ds(base, 128)] = out128.reshape((128,))
        return carry

    lax.fori_loop(0, N_TC // 128, block_body, 0, unroll=False)


_bpr_tc = pl.pallas_call(
    _bpr_tc_body,
    out_shape=jax.ShapeDtypeStruct((N_TC,), jnp.float32),
    in_specs=[
        pl.BlockSpec(memory_space=pltpu.SMEM),
        pl.BlockSpec(memory_space=pltpu.SMEM),
        pl.BlockSpec(memory_space=pltpu.MemorySpace.HBM),
        pl.BlockSpec(memory_space=pltpu.MemorySpace.HBM),
    ],
    out_specs=pl.BlockSpec(memory_space=pltpu.VMEM),
    scratch_shapes=[
        pltpu.VMEM((12, 2, FACTORS, 128), jnp.float32),
        pltpu.VMEM((12, 2, FACTORS, 128), jnp.float32),
    ] + [pltpu.SemaphoreType.DMA] * 12,
)


def kernel(users, item, user_emb_weight, item_emb_weight):
    us = users.astype(jnp.int32)
    it = item.astype(jnp.int32)
    ut = user_emb_weight.T
    itb = item_emb_weight.T
    out_sc = _bpr_sc(us[:N_SC], it[:N_SC], ut, itb)
    out_tc = _bpr_tc(us[N_SC:], it[N_SC:], ut, itb)
    return jnp.concatenate([out_sc, out_tc])


# speedup vs baseline: 1.5115x; 1.5115x over previous
import functools

import jax
import jax.numpy as jnp
from jax import lax
from jax.experimental import pallas as pl
from jax.experimental.pallas import tpu as pltpu
from jax.experimental.pallas import tpu_sc as plsc

BATCH = 16384
FACTORS = 64
NUM_CORES = 2
NUM_SUBCORES = 16
NUM_WORKERS = NUM_CORES * NUM_SUBCORES
N_SC = 13824
N_TC = BATCH - N_SC
BPW = N_SC // NUM_WORKERS
IDX_CHUNK = 16
NCHUNK = BPW // IDX_CHUNK
NGROUPS = BPW // 16
NSUB = 8

_mesh = plsc.VectorSubcoreMesh(core_axis_name="c", subcore_axis_name="s")


@functools.partial(
    pl.kernel,
    out_type=jax.ShapeDtypeStruct((N_SC,), jnp.float32),
    mesh=_mesh,
    compiler_params=pltpu.CompilerParams(needs_layout_passes=False),
    scratch_types=[
        pltpu.VMEM((NCHUNK, IDX_CHUNK), jnp.int32),
        pltpu.VMEM((NCHUNK, IDX_CHUNK), jnp.int32),
        pltpu.VMEM((3, 2, FACTORS, 128), jnp.float32),
        pltpu.VMEM((3, 2, FACTORS, 128), jnp.float32),
        pltpu.VMEM((BPW,), jnp.float32),
        pltpu.SemaphoreType.DMA,
        pltpu.SemaphoreType.DMA,
        pltpu.SemaphoreType.DMA,
        pltpu.SemaphoreType.DMA,
    ],
)
def _bpr_sc(users_hbm, item_hbm, utab_hbm, itab_hbm, out_hbm,
            uidx_v, iidx_v, uwin_v, iwin_v, out_v, sem_idx,
            sem_a, sem_b, sem_c):
    wid = lax.axis_index("s") * NUM_CORES + lax.axis_index("c")
    base = wid * BPW

    for k in range(NCHUNK):
        pltpu.async_copy(users_hbm.at[pl.ds(base + k * IDX_CHUNK, IDX_CHUNK)],
                         uidx_v.at[k], sem_idx)
        pltpu.async_copy(item_hbm.at[pl.ds(base + k * IDX_CHUNK, IDX_CHUNK)],
                         iidx_v.at[k], sem_idx)
    for k in range(NCHUNK):
        pltpu.make_async_copy(users_hbm.at[pl.ds(base, IDX_CHUNK)],
                              uidx_v.at[k], sem_idx).wait()
        pltpu.make_async_copy(item_hbm.at[pl.ds(base, IDX_CHUNK)],
                              iidx_v.at[k], sem_idx).wait()

    lane = lax.iota(jnp.int32, 16)
    sems = (sem_a, sem_b, sem_c)

    def _hsum(v):
        for s in (8, 4, 2, 1):
            v = v + jnp.take_along_axis(v, lane ^ s, axis=0,
                                        mode="promise_in_bounds")
        return v

    def group_body(a, carry):
        uidx16 = uidx_v[a, pl.ds(0, 16)]
        iidx16 = iidx_v[a, pl.ds(0, 16)]
        ucol = (uidx16 >> 7) << 7
        icol = (iidx16 >> 7) << 7
        uoff = uidx16 & 127
        ioff = iidx16 & 127

        def fire(sub):
            slot = sub % 3
            for jj in range(2):
                j = sub * 2 + jj
                uc = pl.multiple_of(ucol[j], 128)
                ic = pl.multiple_of(icol[j], 128)
                pltpu.async_copy(utab_hbm.at[:, pl.ds(uc, 128)],
                                 uwin_v.at[slot, jj], sems[slot])
                pltpu.async_copy(itab_hbm.at[:, pl.ds(ic, 128)],
                                 iwin_v.at[slot, jj], sems[slot])

        def wait(sub):
            slot = sub % 3
            for jj in range(2):
                pltpu.make_async_copy(utab_hbm.at[:, pl.ds(0, 128)],
                                      uwin_v.at[slot, jj], sems[slot]).wait()
                pltpu.make_async_copy(itab_hbm.at[:, pl.ds(0, 128)],
                                      iwin_v.at[slot, jj], sems[slot]).wait()

        out16 = jnp.zeros((16,), jnp.float32)
        fire(0)
        fire(1)
        for sub in range(NSUB):
            if sub + 2 < NSUB:
                fire(sub + 2)
            wait(sub)
            slot = sub % 3
            for jj in range(2):
                j = sub * 2 + jj
                ul = jnp.full((16,), uoff[j], jnp.int32)
                il = jnp.full((16,), ioff[j], jnp.int32)
                acc = None
                for c in range(FACTORS // 16):
                    fidx = lane + (c * 16)
                    ug = plsc.load_gather(uwin_v.at[slot, jj], [fidx, ul])
                    ig = plsc.load_gather(iwin_v.at[slot, jj], [fidx, il])
                    p = ug * ig
                    acc = p if acc is None else acc + p
                out16 = jnp.where(lane == j, _hsum(acc), out16)
        out_v[pl.ds(a * 16, 16)] = out16
        return carry

    lax.fori_loop(0, NGROUPS, group_body, 0, unroll=False)

    pltpu.sync_copy(out_v, out_hbm.at[pl.ds(base, BPW)])


def _bpr_tc_body(uidx_s, iidx_s, utab_hbm, itab_hbm, out_ref,
                 uwin_v, iwin_v, *sems):
    j128 = lax.broadcasted_iota(jnp.int32, (1, 128), 1)
    nsub = 64
    nslot = 12

    def block_body(a, carry):
        def fire(sub):
            slot = sub % nslot
            for jj in range(2):
                r = a * 128 + sub * 2 + jj
                u = uidx_s[r]
                i = iidx_s[r]
                uc = pl.multiple_of((u >> 7) << 7, 128)
                ic = pl.multiple_of((i >> 7) << 7, 128)
                pltpu.make_async_copy(utab_hbm.at[:, pl.ds(uc, 128)],
                                      uwin_v.at[slot, jj], sems[slot]).start()
                pltpu.make_async_copy(itab_hbm.at[:, pl.ds(ic, 128)],
                                      iwin_v.at[slot, jj], sems[slot]).start()

        def wait(sub):
            slot = sub % nslot
            for jj in range(2):
                pltpu.make_async_copy(utab_hbm.at[:, pl.ds(0, 128)],
                                      uwin_v.at[slot, jj], sems[slot]).wait()
                pltpu.make_async_copy(itab_hbm.at[:, pl.ds(0, 128)],
                                      iwin_v.at[slot, jj], sems[slot]).wait()

        out128 = jnp.zeros((1, 128), jnp.float32)
        for s in range(nslot - 1):
            fire(s)
        for sub in range(nsub):
            if sub + nslot - 1 < nsub:
                fire(sub + nslot - 1)
            wait(sub)
            slot = sub % nslot
            for jj in range(2):
                r = a * 128 + sub * 2 + jj
                j = sub * 2 + jj
                cu = uidx_s[r] & 127
                ci = iidx_s[r] & 127
                wu = uwin_v[slot, jj]
                wi = iwin_v[slot, jj]
                gu = jnp.take_along_axis(
                    wu, jnp.full((FACTORS, 1), cu, jnp.int32), axis=1)
                gi = jnp.take_along_axis(
                    wi, jnp.full((FACTORS, 1), ci, jnp.int32), axis=1)
                s = jnp.sum(gu * gi)
                out128 = jnp.where(j128 == j, s, out128)
        base = pl.multiple_of(a * 128, 128)
        out_ref[pl.ds(base, 128)] = out128.reshape((128,))
        return carry

    lax.fori_loop(0, N_TC // 128, block_body, 0, unroll=False)


_bpr_tc = pl.pallas_call(
    _bpr_tc_body,
    out_shape=jax.ShapeDtypeStruct((N_TC,), jnp.float32),
    in_specs=[
        pl.BlockSpec(memory_space=pltpu.SMEM),
        pl.BlockSpec(memory_space=pltpu.SMEM),
        pl.BlockSpec(memory_space=pltpu.MemorySpace.HBM),
        pl.BlockSpec(memory_space=pltpu.MemorySpace.HBM),
    ],
    out_specs=pl.BlockSpec(memory_space=pltpu.VMEM),
    scratch_shapes=[
        pltpu.VMEM((12, 2, FACTORS, 128), jnp.float32),
        pltpu.VMEM((12, 2, FACTORS, 128), jnp.float32),
    ] + [pltpu.SemaphoreType.DMA] * 12,
)


def kernel(users, item, user_emb_weight, item_emb_weight):
    us = users.astype(jnp.int32)
    it = item.astype(jnp.int32)
    ut = user_emb_weight.T
    itb = item_emb_weight.T
    out_sc = _bpr_sc(us[:N_SC], it[:N_SC], ut, itb)
    out_tc = _bpr_tc(us[N_SC:], it[N_SC:], ut, itb)
    return jnp.concatenate([out_sc, out_tc])
